# baseline (speedup 1.0000x reference)
import functools

import jax
import jax.numpy as jnp
from jax import lax
from jax.experimental import pallas as pl
from jax.experimental.pallas import tpu as pltpu
from jax.experimental.pallas import tpu_sc as plsc

F = 26
V = 100000
D = 16
B = 4096
FD = F * D
BF = B * F

_NC, _NS = 2, 16
_NW = _NC * _NS
_N = BF // _NW
_CH = 128
_NCH = _N // _CH


def _sc_gather(emb_flat, lin_wide, idx_2d, lidx_2d):
    mesh = plsc.VectorSubcoreMesh(core_axis_name="c", subcore_axis_name="s")

    @functools.partial(
        pl.kernel,
        mesh=mesh,
        out_type=[
            jax.ShapeDtypeStruct((BF, D), jnp.float32),
            jax.ShapeDtypeStruct((BF, D), jnp.float32),
        ],
        scratch_types=[
            pltpu.VMEM((_NCH, _CH), jnp.int32),
            pltpu.VMEM((_NCH, _CH), jnp.int32),
            pltpu.VMEM((_N, D), jnp.float32),
            pltpu.VMEM((_N, D), jnp.float32),
            pltpu.SemaphoreType.DMA,
            pltpu.SemaphoreType.DMA,
        ],
        compiler_params=pltpu.CompilerParams(use_tc_tiling_on_sc=False),
    )
    def k(emb_hbm, lin_hbm, idx_hbm, lidx_hbm, rows_out, lin_out,
          idx_v, lidx_v, rows_v, lin_v, sem_e, sem_l):
        wid = lax.axis_index("s") * _NC + lax.axis_index("c")
        base = wid * _N
        pltpu.sync_copy(idx_hbm.at[pl.ds(wid * _NCH, _NCH)], idx_v)
        pltpu.sync_copy(lidx_hbm.at[pl.ds(wid * _NCH, _NCH)], lidx_v)
        copies = []
        for j in range(_NCH):
            copies.append(pltpu.async_copy(
                emb_hbm.at[idx_v.at[j]],
                rows_v.at[pl.ds(j * _CH, _CH)], sem_e))
            copies.append(pltpu.async_copy(
                lin_hbm.at[lidx_v.at[j]],
                lin_v.at[pl.ds(j * _CH, _CH)], sem_l))
        for c in copies:
            c.wait()
        pltpu.sync_copy(rows_v, rows_out.at[pl.ds(base, _N)])
        pltpu.sync_copy(lin_v, lin_out.at[pl.ds(base, _N)])

    return k(emb_flat, lin_wide, idx_2d, lidx_2d)


_BT = 512


def _tc_body(deep_ref, linr_ref, cols_ref, w1_ref, b1_ref, w2_ref, b2_ref,
             w3_ref, b3_ref, out_ref):
    x = deep_ref[...]
    linr = linr_ref[...]
    cols = cols_ref[...]

    fidx = lax.broadcasted_iota(jnp.int32, (F, FD), 0)
    jidx = lax.broadcasted_iota(jnp.int32, (F, FD), 1)
    R = (jidx // D == fidx).astype(jnp.float32)
    colv = jax.lax.dot(cols, R)
    jmod = lax.broadcasted_iota(jnp.int32, (_BT, FD), 1) % D
    lin_val = jnp.where(colv == jmod.astype(jnp.float32), linr, 0.0)
    linear_logit = jnp.sum(lin_val, axis=1)

    rows = lax.broadcasted_iota(jnp.int32, (FD, D), 0)
    colsd = lax.broadcasted_iota(jnp.int32, (FD, D), 1)
    S = (rows % D == colsd).astype(jnp.float32)
    s1 = jax.lax.dot(x, S)
    s2 = jax.lax.dot(x * x, S)
    fm_logit = 0.5 * jnp.sum(s1 * s1 - s2, axis=1)

    h = jnp.maximum(jax.lax.dot(x, w1_ref[...]) + b1_ref[...], 0.0)
    h = jnp.maximum(jax.lax.dot(h, w2_ref[...]) + b2_ref[...], 0.0)
    deep = jax.lax.dot(h, w3_ref[...])[:, 0] + b3_ref[0, 0]

    out_ref[...] = linear_logit + fm_logit + deep


def _tc_head(deep_in, lin_rows, cols, W1, b1, W2, b2, W3, b3):
    grid = B // _BT
    return pl.pallas_call(
        _tc_body,
        grid=(grid,),
        in_specs=[
            pl.BlockSpec((_BT, FD), lambda i: (i, 0)),
            pl.BlockSpec((_BT, FD), lambda i: (i, 0)),
            pl.BlockSpec((_BT, F), lambda i: (i, 0)),
            pl.BlockSpec((FD, 64), lambda i: (0, 0)),
            pl.BlockSpec((1, 64), lambda i: (0, 0)),
            pl.BlockSpec((64, 32), lambda i: (0, 0)),
            pl.BlockSpec((1, 32), lambda i: (0, 0)),
            pl.BlockSpec((32, 1), lambda i: (0, 0)),
            pl.BlockSpec((1, 1), lambda i: (0, 0)),
        ],
        out_specs=pl.BlockSpec((_BT,), lambda i: (i,)),
        out_shape=jax.ShapeDtypeStruct((B,), jnp.float32),
    )(deep_in, lin_rows, cols, W1, b1, W2, b2, W3, b3)


def kernel(x_cat, lin_tables, emb_tables, W1, b1, W2, b2, W3, b3):
    xc = x_cat.astype(jnp.int32)
    idx = (xc + jnp.arange(F, dtype=jnp.int32)[None, :] * V)
    lidx = idx // D
    cols = (xc % D).astype(jnp.float32)

    emb_flat = emb_tables.reshape(F * V, D)
    lin_wide = lin_tables.reshape(F * V // D, D)

    rows, lin_rows = _sc_gather(emb_flat, lin_wide,
                                idx.reshape(BF // _CH, _CH),
                                lidx.reshape(BF // _CH, _CH))
    return _tc_head(rows.reshape(B, FD), lin_rows.reshape(B, FD), cols,
                    W1, b1.reshape(1, 64), W2, b2.reshape(1, 32),
                    W3, b3.reshape(1, 1))


# speedup vs baseline: 7.3823x; 7.3823x over previous
import functools

import jax
import jax.numpy as jnp
from jax import lax
from jax.experimental import pallas as pl
from jax.experimental.pallas import tpu as pltpu
from jax.experimental.pallas import tpu_sc as plsc

F = 26
V = 100000
D = 16
B = 4096
FD = F * D

_NC, _NS = 2, 16


def _sc_emb_select(emb_t, x_t):
    mesh = plsc.VectorSubcoreMesh(core_axis_name="c", subcore_axis_name="s")

    @functools.partial(
        pl.kernel,
        mesh=mesh,
        out_type=jax.ShapeDtypeStruct((FD, B), jnp.float32),
        scratch_types=[
            pltpu.VMEM((V,), jnp.float32),
            pltpu.VMEM((B,), jnp.int32),
            pltpu.VMEM((B,), jnp.float32),
        ],
        compiler_params=pltpu.CompilerParams(
            use_tc_tiling_on_sc=True, needs_layout_passes=False),
    )
    def k(emb_hbm, x_hbm, out_hbm, plane_v, idx_v, sel_v):
        c = lax.axis_index("c")
        s = lax.axis_index("s")
        for kf in range(F // _NC):
            f = 2 * kf
            fdyn = f + c
            pltpu.sync_copy(x_hbm.at[fdyn], idx_v)
            pltpu.sync_copy(emb_hbm.at[fdyn, s], plane_v)

            def body(i, _):
                v16 = idx_v[pl.ds(i * 16, 16)]
                sel_v[pl.ds(i * 16, 16)] = plsc.load_gather(plane_v, [v16])
                return 0

            lax.fori_loop(0, B // 16, body, 0)
            pltpu.sync_copy(sel_v, out_hbm.at[fdyn * D + s])

    return k(emb_t, x_t)


def _sc_lin_select(lin_t, x_t):
    mesh = plsc.VectorSubcoreMesh(core_axis_name="c", subcore_axis_name="s")

    @functools.partial(
        pl.kernel,
        mesh=mesh,
        out_type=jax.ShapeDtypeStruct((F, B), jnp.float32),
        scratch_types=[
            pltpu.VMEM((V,), jnp.float32),
            pltpu.VMEM((B,), jnp.int32),
            pltpu.VMEM((B,), jnp.float32),
        ],
        compiler_params=pltpu.CompilerParams(
            use_tc_tiling_on_sc=True, needs_layout_passes=False),
    )
    def k(lin_hbm, x_hbm, out_hbm, plane_v, idx_v, sel_v):
        wid = lax.axis_index("s") * _NC + lax.axis_index("c")

        @pl.when(wid < F)
        def _():
            pltpu.sync_copy(x_hbm.at[wid], idx_v)
            pltpu.sync_copy(lin_hbm.at[wid], plane_v)

            def body(i, _):
                v16 = idx_v[pl.ds(i * 16, 16)]
                sel_v[pl.ds(i * 16, 16)] = plsc.load_gather(plane_v, [v16])
                return 0

            lax.fori_loop(0, B // 16, body, 0)
            pltpu.sync_copy(sel_v, out_hbm.at[wid])

    return k(lin_t, x_t)


_BT = 1024


def _tc_body(xt_ref, lt_ref, w1_ref, b1_ref, w2_ref, b2_ref, w3_ref, b3_ref,
             out_ref):
    xt = xt_ref[...]
    lt = lt_ref[...]
    linear_logit = jnp.sum(lt, axis=0)

    didx = lax.broadcasted_iota(jnp.int32, (D, FD), 0)
    ridx = lax.broadcasted_iota(jnp.int32, (D, FD), 1)
    R = (ridx % D == didx).astype(jnp.float32)
    dn = (((1,), (0,)), ((), ()))
    s1 = lax.dot_general(R, xt, dn)
    s2 = lax.dot_general(R, xt * xt, dn)
    fm_logit = 0.5 * jnp.sum(s1 * s1 - s2, axis=0)

    dnT = (((0,), (0,)), ((), ()))
    h = jnp.maximum(lax.dot_general(w1_ref[...], xt, dnT) + b1_ref[...], 0.0)
    h = jnp.maximum(lax.dot_general(w2_ref[...], h, dnT) + b2_ref[...], 0.0)
    deep = lax.dot_general(w3_ref[...], h, dnT)[0, :] + b3_ref[0, 0]

    out_ref[...] = linear_logit + fm_logit + deep


def _tc_head(xt, lt, W1, b1, W2, b2, W3, b3):
    grid = B // _BT
    return pl.pallas_call(
        _tc_body,
        grid=(grid,),
        in_specs=[
            pl.BlockSpec((FD, _BT), lambda i: (0, i)),
            pl.BlockSpec((F, _BT), lambda i: (0, i)),
            pl.BlockSpec((FD, 64), lambda i: (0, 0)),
            pl.BlockSpec((64, 1), lambda i: (0, 0)),
            pl.BlockSpec((64, 32), lambda i: (0, 0)),
            pl.BlockSpec((32, 1), lambda i: (0, 0)),
            pl.BlockSpec((32, 1), lambda i: (0, 0)),
            pl.BlockSpec((1, 1), lambda i: (0, 0)),
        ],
        out_specs=pl.BlockSpec((_BT,), lambda i: (i,)),
        out_shape=jax.ShapeDtypeStruct((B,), jnp.float32),
    )(xt, lt, W1, b1, W2, b2, W3, b3)


def kernel(x_cat, lin_tables, emb_tables, W1, b1, W2, b2, W3, b3):
    emb_t = jnp.transpose(emb_tables, (0, 2, 1))
    lin_t = jnp.transpose(lin_tables, (0, 2, 1)).reshape(F, V)
    x_t = jnp.transpose(x_cat.astype(jnp.int32), (1, 0))

    xt = _sc_emb_select(emb_t, x_t)
    lt = _sc_lin_select(lin_t, x_t)

    return _tc_head(xt, lt, W1, b1.reshape(64, 1), W2, b2.reshape(32, 1),
                    W3, b3.reshape(1, 1))
